# baseline (speedup 1.0000x reference)
import functools

import jax
import jax.numpy as jnp
from jax import lax
from jax.experimental import pallas as pl
from jax.experimental.pallas import tpu as pltpu
from jax.experimental.pallas import tpu_sc as plsc

_B = 32
_D = 4096
_H = 16384
_K = 64
_TH = 1024

_NC = 2
_NS = 16
_L = 16
_CAP = 144
_GS = 8
_GRP = _K // _GS


def _encode_body(x_ref, w_ref, benc_ref, bpre_ref, pre_ref, mu_ref, std_ref, xn_ref):
    j = pl.program_id(0)

    @pl.when(j == 0)
    def _():
        xv = x_ref[...]
        mu = jnp.mean(xv, axis=1, keepdims=True)
        diff = xv - mu
        var = jnp.sum(diff * diff, axis=1, keepdims=True) * (1.0 / (_D - 1))
        std = jnp.sqrt(var)
        mu_ref[...] = mu
        std_ref[...] = std
        xn_ref[...] = diff / (std + 1e-5) - bpre_ref[...]

    pre_ref[...] = (
        jnp.dot(xn_ref[...], w_ref[...], preferred_element_type=jnp.float32)
        + benc_ref[...]
    )


def _search_body(pre_ref, t_ref):
    v = pre_ref[...]
    cpos = jnp.sum((v > 0.0).astype(jnp.int32), axis=1, keepdims=True)

    def body(_, carry):
        lo, hi = carry
        mid = lo + (hi - lo) // 2
        tm = lax.bitcast_convert_type(mid, jnp.float32)
        cnt = jnp.sum((v >= tm).astype(jnp.int32), axis=1, keepdims=True)
        ge = cnt >= _K
        return jnp.where(ge, mid, lo), jnp.where(ge, hi, mid)

    lo0 = jnp.zeros((_B, 1), jnp.int32)
    hi0 = jnp.full((_B, 1), 0x7F800001, jnp.int32)
    lo, _ = lax.fori_loop(0, 31, body, (lo0, hi0))
    t = lax.bitcast_convert_type(lo, jnp.float32)
    t_ref[...] = jnp.where(cpos > _K, t, 0.0)


def _popcount_scalar(m):
    pcv = plsc.all_reduce_population_count(m)
    return lax.squeeze(lax.slice(pcv, (0,), (1,)), (0,))


def _lane_splat(chunk, lane_idx_vec):
    dn = lax.GatherDimensionNumbers(
        offset_dims=(), collapsed_slice_dims=(0,), start_index_map=(0,))
    return lax.gather(chunk, lane_idx_vec[:, None], dn, (1,),
                      mode=lax.GatherScatterMode.PROMISE_IN_BOUNDS)


def _row_splat(ref_v, base, widv):
    lane = jnp.bitwise_and(widv, _L - 1)
    s0 = _lane_splat(ref_v[pl.ds(base, _L)], lane)
    s1 = _lane_splat(ref_v[pl.ds(base + _L, _L)], lane)
    return jnp.where(widv < _L, s0, s1)


def _sc_decode_body(pre_hbm, aux_hbm, bpre_hbm, wdec_hbm,
                    out_hbm,
                    row_v, aux_v, ei, si, acc,
                    rows0, rows1, sem0, sem1, sem2):
    wid = lax.axis_index("s") * _NC + lax.axis_index("c")
    d_row = pltpu.async_copy(pre_hbm.at[wid], row_v, sem0)
    d_aux = pltpu.async_copy(aux_hbm, aux_v, sem1)
    d_acc = pltpu.async_copy(bpre_hbm, acc, sem2)

    zi = jnp.zeros((_L,), jnp.int32)
    for c in range(_CAP // _L):
        si[pl.ds(c * _L, _L)] = zi
    laneiota = lax.iota(jnp.int32, _L)

    d_aux.wait()
    widv = jnp.zeros((_L,), jnp.int32) + wid
    ts = _row_splat(aux_v, 0, widv)
    pos_t = jnp.maximum(ts, 0.0)
    d_row.wait()

    def scan_body(i, carry):
        ns, ne = carry
        v = row_v[pl.ds(i * _L, _L)]
        idx = laneiota + i * _L
        m = v > pos_t
        plsc.store_compressed(si.at[pl.ds(ns, _L)], idx, mask=m)
        ns = ns + _popcount_scalar(m)
        meq = jnp.logical_and(v == pos_t, ne <= _K)
        plsc.store_compressed(ei.at[pl.ds(ne, _L)], idx, mask=meq)
        ne = ne + _popcount_scalar(meq)
        return ns, ne

    ns, ne = lax.fori_loop(0, _H // _L, scan_body,
                           (jnp.int32(0), jnp.int32(0)), unroll=8)

    need = jnp.minimum(_K - ns, ne)
    total = ns + need
    needv = jnp.zeros((_L,), jnp.int32) + need
    totalv = jnp.zeros((_L,), jnp.int32) + total
    for c in range(_K // _L):
        lane = laneiota + c * _L
        keep = lane < needv
        plsc.store_compressed(si.at[pl.ds(ns + c * _L, _L)],
                              ei[pl.ds(c * _L, _L)], mask=keep)

    bufs = [rows0, rows1]
    sems = [sem0, sem1]
    descs = [None] * _GRP
    descs[0] = pltpu.async_copy(
        wdec_hbm.at[si.at[pl.ds(0, _GS)]], bufs[0], sems[0])
    d_acc.wait()
    vchunk = None
    for g in range(_GRP):
        if g + 1 < _GRP:
            descs[g + 1] = pltpu.async_copy(
                wdec_hbm.at[si.at[pl.ds((g + 1) * _GS, _GS)]],
                bufs[(g + 1) % 2], sems[(g + 1) % 2])
        if g % 2 == 0:
            slot = laneiota + (g // 2) * _L
            ichunk = si[pl.ds((g // 2) * _L, _L)]
            vchunk = jnp.where(slot < totalv,
                               plsc.load_gather(row_v, [ichunk]), 0.0)
        descs[g].wait()
        cur = bufs[g % 2]
        vsp = [_lane_splat(vchunk, jnp.full((_L,), (g % 2) * _GS + r, jnp.int32))
               for r in range(_GS)]

        @plsc.parallel_loop(0, _D // _L, 1, unroll=8)
        def _mac(c):
            base = c * _L
            a = acc[pl.ds(base, _L)]
            for r in range(_GS):
                a = a + vsp[r] * cur[r, pl.ds(base, _L)]
            acc[pl.ds(base, _L)] = a

    mus = _row_splat(aux_v, _B, widv)
    sds = _row_splat(aux_v, 2 * _B, widv)

    @plsc.parallel_loop(0, _D // _L, 1, unroll=4)
    def _fin(c):
        base = c * _L
        acc[pl.ds(base, _L)] = acc[pl.ds(base, _L)] * sds + mus

    pltpu.sync_copy(acc, out_hbm.at[wid])


def kernel(x, w_enc, w_dec, b_enc, b_pre):
    pre, mu, std = pl.pallas_call(
        _encode_body,
        grid=(_H // _TH,),
        in_specs=[
            pl.BlockSpec((_B, _D), lambda j: (0, 0)),
            pl.BlockSpec((_D, _TH), lambda j: (0, j)),
            pl.BlockSpec((_TH,), lambda j: (j,)),
            pl.BlockSpec((_D,), lambda j: (0,)),
        ],
        out_specs=[
            pl.BlockSpec((_B, _TH), lambda j: (0, j)),
            pl.BlockSpec((_B, 1), lambda j: (0, 0)),
            pl.BlockSpec((_B, 1), lambda j: (0, 0)),
        ],
        out_shape=[
            jax.ShapeDtypeStruct((_B, _H), jnp.float32),
            jax.ShapeDtypeStruct((_B, 1), jnp.float32),
            jax.ShapeDtypeStruct((_B, 1), jnp.float32),
        ],
        scratch_shapes=[pltpu.VMEM((_B, _D), jnp.float32)],
    )(x, w_enc, b_enc, b_pre)

    t = pl.pallas_call(
        _search_body,
        out_shape=jax.ShapeDtypeStruct((_B, 1), jnp.float32),
    )(pre)

    sc_decode = pl.kernel(
        _sc_decode_body,
        out_type=jax.ShapeDtypeStruct((_B, _D), jnp.float32),
        mesh=plsc.VectorSubcoreMesh(core_axis_name="c", subcore_axis_name="s",
                                    num_cores=_NC, num_subcores=_NS),
        scratch_types=[
            pltpu.VMEM((_H,), jnp.float32),
            pltpu.VMEM((3 * _B,), jnp.float32),
            pltpu.VMEM((112,), jnp.int32),
            pltpu.VMEM((_CAP,), jnp.int32),
            pltpu.VMEM((_D,), jnp.float32),
            pltpu.VMEM((_GS, _D), jnp.float32),
            pltpu.VMEM((_GS, _D), jnp.float32),
            pltpu.SemaphoreType.DMA,
            pltpu.SemaphoreType.DMA,
            pltpu.SemaphoreType.DMA,
        ],
        compiler_params=pltpu.CompilerParams(needs_layout_passes=False),
    )
    aux = jnp.concatenate([t.reshape(_B), mu.reshape(_B), std.reshape(_B)])
    recons = sc_decode(pre, aux, b_pre, w_dec)

    num_dead = jnp.zeros((), jnp.int32)
    return (recons, num_dead)


# speedup vs baseline: 1.0209x; 1.0209x over previous
import functools

import jax
import jax.numpy as jnp
from jax import lax
from jax.experimental import pallas as pl
from jax.experimental.pallas import tpu as pltpu
from jax.experimental.pallas import tpu_sc as plsc

_B = 32
_D = 4096
_H = 16384
_K = 64
_TH = 1024

_NC = 2
_NS = 16
_L = 16
_CAP = 144
_GS = 8
_GRP = _K // _GS


def _encode_body(x_ref, w_ref, benc_ref, bpre_ref, pre_ref, mu_ref, std_ref,
                 t_ref, xn_ref, pre_all):
    j = pl.program_id(0)

    @pl.when(j == 0)
    def _():
        xv = x_ref[...]
        mu = jnp.mean(xv, axis=1, keepdims=True)
        diff = xv - mu
        var = jnp.sum(diff * diff, axis=1, keepdims=True) * (1.0 / (_D - 1))
        std = jnp.sqrt(var)
        mu_ref[...] = mu
        std_ref[...] = std
        xn_ref[...] = diff / (std + 1e-5) - bpre_ref[...]

    tile = (
        jnp.dot(xn_ref[...], w_ref[...], preferred_element_type=jnp.float32)
        + benc_ref[...]
    )
    pre_ref[...] = tile
    pre_all[:, pl.ds(j * _TH, _TH)] = tile

    @pl.when(j == pl.num_programs(0) - 1)
    def _():
        v = pre_all[...]
        cpos = jnp.sum((v > 0.0).astype(jnp.int32), axis=1, keepdims=True)

        def body(_, carry):
            lo, hi = carry
            mid = lo + (hi - lo) // 2
            tm = lax.bitcast_convert_type(mid, jnp.float32)
            cnt = jnp.sum((v >= tm).astype(jnp.int32), axis=1, keepdims=True)
            ge = cnt >= _K
            return jnp.where(ge, mid, lo), jnp.where(ge, hi, mid)

        lo0 = jnp.zeros((_B, 1), jnp.int32)
        hi0 = jnp.full((_B, 1), 0x7F800001, jnp.int32)
        lo, _ = lax.fori_loop(0, 31, body, (lo0, hi0))
        t = lax.bitcast_convert_type(lo, jnp.float32)
        t_ref[...] = jnp.where(cpos > _K, t, 0.0)


def _popcount_scalar(m):
    pcv = plsc.all_reduce_population_count(m)
    return lax.squeeze(lax.slice(pcv, (0,), (1,)), (0,))


def _lane_splat(chunk, lane_idx_vec):
    dn = lax.GatherDimensionNumbers(
        offset_dims=(), collapsed_slice_dims=(0,), start_index_map=(0,))
    return lax.gather(chunk, lane_idx_vec[:, None], dn, (1,),
                      mode=lax.GatherScatterMode.PROMISE_IN_BOUNDS)


def _row_splat(ref_v, base, widv):
    lane = jnp.bitwise_and(widv, _L - 1)
    s0 = _lane_splat(ref_v[pl.ds(base, _L)], lane)
    s1 = _lane_splat(ref_v[pl.ds(base + _L, _L)], lane)
    return jnp.where(widv < _L, s0, s1)


def _sc_decode_body(pre_hbm, aux_hbm, bpre_hbm, wdec_hbm,
                    out_hbm,
                    row_v, aux_v, ei, si, acc,
                    rows0, rows1, sem0, sem1, sem2):
    wid = lax.axis_index("s") * _NC + lax.axis_index("c")
    d_row = pltpu.async_copy(pre_hbm.at[wid], row_v, sem0)
    d_aux = pltpu.async_copy(aux_hbm, aux_v, sem1)
    d_acc = pltpu.async_copy(bpre_hbm, acc, sem2)

    zi = jnp.zeros((_L,), jnp.int32)
    for c in range(_CAP // _L):
        si[pl.ds(c * _L, _L)] = zi
    laneiota = lax.iota(jnp.int32, _L)

    d_aux.wait()
    widv = jnp.zeros((_L,), jnp.int32) + wid
    ts = _row_splat(aux_v, 0, widv)
    pos_t = jnp.maximum(ts, 0.0)
    d_row.wait()

    def scan_body(i, carry):
        ns, ne = carry
        v = row_v[pl.ds(i * _L, _L)]
        idx = laneiota + i * _L
        m = v > pos_t
        plsc.store_compressed(si.at[pl.ds(ns, _L)], idx, mask=m)
        ns = ns + _popcount_scalar(m)
        meq = jnp.logical_and(v == pos_t, ne <= _K)
        plsc.store_compressed(ei.at[pl.ds(ne, _L)], idx, mask=meq)
        ne = ne + _popcount_scalar(meq)
        return ns, ne

    ns, ne = lax.fori_loop(0, _H // _L, scan_body,
                           (jnp.int32(0), jnp.int32(0)), unroll=8)

    need = jnp.minimum(_K - ns, ne)
    total = ns + need
    needv = jnp.zeros((_L,), jnp.int32) + need
    totalv = jnp.zeros((_L,), jnp.int32) + total
    for c in range(_K // _L):
        lane = laneiota + c * _L
        keep = lane < needv
        plsc.store_compressed(si.at[pl.ds(ns + c * _L, _L)],
                              ei[pl.ds(c * _L, _L)], mask=keep)

    bufs = [rows0, rows1]
    sems = [sem0, sem1]
    descs = [None] * _GRP
    descs[0] = pltpu.async_copy(
        wdec_hbm.at[si.at[pl.ds(0, _GS)]], bufs[0], sems[0])
    d_acc.wait()
    vchunk = None
    for g in range(_GRP):
        if g + 1 < _GRP:
            descs[g + 1] = pltpu.async_copy(
                wdec_hbm.at[si.at[pl.ds((g + 1) * _GS, _GS)]],
                bufs[(g + 1) % 2], sems[(g + 1) % 2])
        if g % 2 == 0:
            slot = laneiota + (g // 2) * _L
            ichunk = si[pl.ds((g // 2) * _L, _L)]
            vchunk = jnp.where(slot < totalv,
                               plsc.load_gather(row_v, [ichunk]), 0.0)
        descs[g].wait()
        cur = bufs[g % 2]
        vsp = [_lane_splat(vchunk, jnp.full((_L,), (g % 2) * _GS + r, jnp.int32))
               for r in range(_GS)]

        @plsc.parallel_loop(0, _D // _L, 1, unroll=8)
        def _mac(c):
            base = c * _L
            a = acc[pl.ds(base, _L)]
            for r in range(_GS):
                a = a + vsp[r] * cur[r, pl.ds(base, _L)]
            acc[pl.ds(base, _L)] = a

    mus = _row_splat(aux_v, _B, widv)
    sds = _row_splat(aux_v, 2 * _B, widv)

    @plsc.parallel_loop(0, _D // _L, 1, unroll=4)
    def _fin(c):
        base = c * _L
        acc[pl.ds(base, _L)] = acc[pl.ds(base, _L)] * sds + mus

    pltpu.sync_copy(acc, out_hbm.at[wid])


def kernel(x, w_enc, w_dec, b_enc, b_pre):
    pre, mu, std, t = pl.pallas_call(
        _encode_body,
        grid=(_H // _TH,),
        in_specs=[
            pl.BlockSpec((_B, _D), lambda j: (0, 0)),
            pl.BlockSpec((_D, _TH), lambda j: (0, j)),
            pl.BlockSpec((_TH,), lambda j: (j,)),
            pl.BlockSpec((_D,), lambda j: (0,)),
        ],
        out_specs=[
            pl.BlockSpec((_B, _TH), lambda j: (0, j)),
            pl.BlockSpec((_B, 1), lambda j: (0, 0)),
            pl.BlockSpec((_B, 1), lambda j: (0, 0)),
            pl.BlockSpec((_B, 1), lambda j: (0, 0)),
        ],
        out_shape=[
            jax.ShapeDtypeStruct((_B, _H), jnp.float32),
            jax.ShapeDtypeStruct((_B, 1), jnp.float32),
            jax.ShapeDtypeStruct((_B, 1), jnp.float32),
            jax.ShapeDtypeStruct((_B, 1), jnp.float32),
        ],
        scratch_shapes=[
            pltpu.VMEM((_B, _D), jnp.float32),
            pltpu.VMEM((_B, _H), jnp.float32),
        ],
    )(x, w_enc, b_enc, b_pre)

    sc_decode = pl.kernel(
        _sc_decode_body,
        out_type=jax.ShapeDtypeStruct((_B, _D), jnp.float32),
        mesh=plsc.VectorSubcoreMesh(core_axis_name="c", subcore_axis_name="s",
                                    num_cores=_NC, num_subcores=_NS),
        scratch_types=[
            pltpu.VMEM((_H,), jnp.float32),
            pltpu.VMEM((3 * _B,), jnp.float32),
            pltpu.VMEM((112,), jnp.int32),
            pltpu.VMEM((_CAP,), jnp.int32),
            pltpu.VMEM((_D,), jnp.float32),
            pltpu.VMEM((_GS, _D), jnp.float32),
            pltpu.VMEM((_GS, _D), jnp.float32),
            pltpu.SemaphoreType.DMA,
            pltpu.SemaphoreType.DMA,
            pltpu.SemaphoreType.DMA,
        ],
        compiler_params=pltpu.CompilerParams(needs_layout_passes=False),
    )
    aux = jnp.concatenate([t.reshape(_B), mu.reshape(_B), std.reshape(_B)])
    recons = sc_decode(pre, aux, b_pre, w_dec)

    num_dead = jnp.zeros((), jnp.int32)
    return (recons, num_dead)


# speedup vs baseline: 1.0371x; 1.0159x over previous
import functools

import jax
import jax.numpy as jnp
from jax import lax
from jax.experimental import pallas as pl
from jax.experimental.pallas import tpu as pltpu
from jax.experimental.pallas import tpu_sc as plsc

_B = 32
_D = 4096
_H = 16384
_K = 64
_TH = 1024

_NC = 2
_NS = 16
_L = 16
_CAP = 144
_GS = 8
_GRP = _K // _GS


def _encode_body(x_ref, w_ref, benc_ref, bpre_ref, pre_ref, mu_ref, std_ref,
                 t_ref, xn_ref, pre_all):
    j = pl.program_id(0)

    @pl.when(j == 0)
    def _():
        xv = x_ref[...]
        mu = jnp.mean(xv, axis=1, keepdims=True)
        diff = xv - mu
        var = jnp.sum(diff * diff, axis=1, keepdims=True) * (1.0 / (_D - 1))
        std = jnp.sqrt(var)
        mu_ref[...] = mu
        std_ref[...] = std
        xn_ref[...] = diff / (std + 1e-5) - bpre_ref[...]

    tile = (
        jnp.dot(xn_ref[...], w_ref[...], preferred_element_type=jnp.float32)
        + benc_ref[...]
    )
    pre_ref[...] = tile
    pre_all[:, pl.ds(j * _TH, _TH)] = tile

    @pl.when(j == pl.num_programs(0) - 1)
    def _():
        v = pre_all[...]
        cpos = jnp.sum((v > 0.0).astype(jnp.int32), axis=1, keepdims=True)

        def body(_, carry):
            lo, hi = carry
            mid = lo + (hi - lo) // 2
            tm = lax.bitcast_convert_type(mid, jnp.float32)
            cnt = jnp.sum((v >= tm).astype(jnp.int32), axis=1, keepdims=True)
            ge = cnt >= _K
            return jnp.where(ge, mid, lo), jnp.where(ge, hi, mid)

        lo0 = jnp.zeros((_B, 1), jnp.int32)
        hi0 = jnp.full((_B, 1), 0x7F800001, jnp.int32)
        lo, _ = lax.fori_loop(0, 31, body, (lo0, hi0))
        t = lax.bitcast_convert_type(lo, jnp.float32)
        t_ref[...] = jnp.where(cpos > _K, t, 0.0)


def _popcount_scalar(m):
    pcv = plsc.all_reduce_population_count(m)
    return lax.squeeze(lax.slice(pcv, (0,), (1,)), (0,))


def _lane_splat(chunk, lane_idx_vec):
    dn = lax.GatherDimensionNumbers(
        offset_dims=(), collapsed_slice_dims=(0,), start_index_map=(0,))
    return lax.gather(chunk, lane_idx_vec[:, None], dn, (1,),
                      mode=lax.GatherScatterMode.PROMISE_IN_BOUNDS)


def _row_splat(ref_v, base, widv):
    lane = jnp.bitwise_and(widv, _L - 1)
    s0 = _lane_splat(ref_v[pl.ds(base, _L)], lane)
    s1 = _lane_splat(ref_v[pl.ds(base + _L, _L)], lane)
    return jnp.where(widv < _L, s0, s1)


def _sc_decode_body(pre_hbm, aux_hbm, bpre_hbm, wdec_hbm,
                    out_hbm,
                    row_v, aux_v, ei, si, acc,
                    rows0, rows1, sem0, sem1, sem2):
    wid = lax.axis_index("s") * _NC + lax.axis_index("c")
    d_row = pltpu.async_copy(pre_hbm.at[wid], row_v, sem0)
    d_aux = pltpu.async_copy(aux_hbm, aux_v, sem1)
    d_acc = pltpu.async_copy(bpre_hbm, acc, sem2)

    zi = jnp.zeros((_L,), jnp.int32)
    for c in range(_CAP // _L):
        si[pl.ds(c * _L, _L)] = zi
    laneiota = lax.iota(jnp.int32, _L)

    d_aux.wait()
    widv = jnp.zeros((_L,), jnp.int32) + wid
    ts = _row_splat(aux_v, 0, widv)
    pos_t = jnp.maximum(ts, 0.0)
    d_row.wait()

    def fast_body(i, nc):
        v = row_v[pl.ds(i * _L, _L)]
        m = jnp.logical_and(v >= pos_t, nc <= _K)
        plsc.store_compressed(si.at[pl.ds(jnp.minimum(nc, _K + _L), _L)],
                              laneiota + i * _L, mask=m)
        return nc + _popcount_scalar(v >= pos_t)

    nc = lax.fori_loop(0, _H // _L, fast_body, jnp.int32(0), unroll=8)

    def slow_path():
        def scan_body(i, carry):
            ns, ne = carry
            v = row_v[pl.ds(i * _L, _L)]
            idx = laneiota + i * _L
            m = v > pos_t
            plsc.store_compressed(si.at[pl.ds(ns, _L)], idx, mask=m)
            ns = ns + _popcount_scalar(m)
            meq = jnp.logical_and(v == pos_t, ne <= _K)
            plsc.store_compressed(ei.at[pl.ds(ne, _L)], idx, mask=meq)
            ne = ne + _popcount_scalar(meq)
            return ns, ne

        ns, ne = lax.fori_loop(0, _H // _L, scan_body,
                               (jnp.int32(0), jnp.int32(0)), unroll=4)
        need = jnp.minimum(_K - ns, ne)
        needv = jnp.zeros((_L,), jnp.int32) + need
        for c in range(_K // _L):
            keep = (laneiota + c * _L) < needv
            plsc.store_compressed(si.at[pl.ds(ns + c * _L, _L)],
                                  ei[pl.ds(c * _L, _L)], mask=keep)
        return ns + need

    total = lax.cond(nc == _K, lambda: jnp.int32(_K), slow_path)
    totalv = jnp.zeros((_L,), jnp.int32) + total

    bufs = [rows0, rows1]
    sems = [sem0, sem1]
    descs = [None] * _GRP
    descs[0] = pltpu.async_copy(
        wdec_hbm.at[si.at[pl.ds(0, _GS)]], bufs[0], sems[0])
    d_acc.wait()
    vchunk = None
    for g in range(_GRP):
        if g + 1 < _GRP:
            descs[g + 1] = pltpu.async_copy(
                wdec_hbm.at[si.at[pl.ds((g + 1) * _GS, _GS)]],
                bufs[(g + 1) % 2], sems[(g + 1) % 2])
        if g % 2 == 0:
            slot = laneiota + (g // 2) * _L
            ichunk = si[pl.ds((g // 2) * _L, _L)]
            vchunk = jnp.where(slot < totalv,
                               plsc.load_gather(row_v, [ichunk]), 0.0)
        descs[g].wait()
        cur = bufs[g % 2]
        vsp = [_lane_splat(vchunk, jnp.full((_L,), (g % 2) * _GS + r, jnp.int32))
               for r in range(_GS)]

        @plsc.parallel_loop(0, _D // _L, 1, unroll=8)
        def _mac(c):
            base = c * _L
            a = acc[pl.ds(base, _L)]
            for r in range(_GS):
                a = a + vsp[r] * cur[r, pl.ds(base, _L)]
            acc[pl.ds(base, _L)] = a

    mus = _row_splat(aux_v, _B, widv)
    sds = _row_splat(aux_v, 2 * _B, widv)

    @plsc.parallel_loop(0, _D // _L, 1, unroll=4)
    def _fin(c):
        base = c * _L
        acc[pl.ds(base, _L)] = acc[pl.ds(base, _L)] * sds + mus

    pltpu.sync_copy(acc, out_hbm.at[wid])


def kernel(x, w_enc, w_dec, b_enc, b_pre):
    pre, mu, std, t = pl.pallas_call(
        _encode_body,
        grid=(_H // _TH,),
        in_specs=[
            pl.BlockSpec((_B, _D), lambda j: (0, 0)),
            pl.BlockSpec((_D, _TH), lambda j: (0, j)),
            pl.BlockSpec((_TH,), lambda j: (j,)),
            pl.BlockSpec((_D,), lambda j: (0,)),
        ],
        out_specs=[
            pl.BlockSpec((_B, _TH), lambda j: (0, j)),
            pl.BlockSpec((_B, 1), lambda j: (0, 0)),
            pl.BlockSpec((_B, 1), lambda j: (0, 0)),
            pl.BlockSpec((_B, 1), lambda j: (0, 0)),
        ],
        out_shape=[
            jax.ShapeDtypeStruct((_B, _H), jnp.float32),
            jax.ShapeDtypeStruct((_B, 1), jnp.float32),
            jax.ShapeDtypeStruct((_B, 1), jnp.float32),
            jax.ShapeDtypeStruct((_B, 1), jnp.float32),
        ],
        scratch_shapes=[
            pltpu.VMEM((_B, _D), jnp.float32),
            pltpu.VMEM((_B, _H), jnp.float32),
        ],
    )(x, w_enc, b_enc, b_pre)

    sc_decode = pl.kernel(
        _sc_decode_body,
        out_type=jax.ShapeDtypeStruct((_B, _D), jnp.float32),
        mesh=plsc.VectorSubcoreMesh(core_axis_name="c", subcore_axis_name="s",
                                    num_cores=_NC, num_subcores=_NS),
        scratch_types=[
            pltpu.VMEM((_H,), jnp.float32),
            pltpu.VMEM((3 * _B,), jnp.float32),
            pltpu.VMEM((112,), jnp.int32),
            pltpu.VMEM((_CAP,), jnp.int32),
            pltpu.VMEM((_D,), jnp.float32),
            pltpu.VMEM((_GS, _D), jnp.float32),
            pltpu.VMEM((_GS, _D), jnp.float32),
            pltpu.SemaphoreType.DMA,
            pltpu.SemaphoreType.DMA,
            pltpu.SemaphoreType.DMA,
        ],
        compiler_params=pltpu.CompilerParams(needs_layout_passes=False),
    )
    aux = jnp.concatenate([t.reshape(_B), mu.reshape(_B), std.reshape(_B)])
    recons = sc_decode(pre, aux, b_pre, w_dec)

    num_dead = jnp.zeros((), jnp.int32)
    return (recons, num_dead)


# speedup vs baseline: 1.0407x; 1.0035x over previous
import jax
import jax.numpy as jnp
from jax import lax
from jax.experimental import pallas as pl
from jax.experimental.pallas import tpu as pltpu
from jax.experimental.pallas import tpu_sc as plsc

_B = 32
_D = 4096
_H = 16384
_K = 64
_TH = 1024

_NC = 2
_NS = 16
_L = 16
_CAP = 144
_GS = 8
_GRP = _K // _GS


def _encode_body(x_ref, w_ref, benc_ref, bpre_ref, pre_ref, mu_ref, std_ref,
                 t_ref, xn_ref, pre_all):
    j = pl.program_id(0)

    @pl.when(j == 0)
    def _():
        xv = x_ref[...]
        mu = jnp.mean(xv, axis=1, keepdims=True)
        diff = xv - mu
        var = jnp.sum(diff * diff, axis=1, keepdims=True) * (1.0 / (_D - 1))
        std = jnp.sqrt(var)
        mu_ref[...] = mu
        std_ref[...] = std
        xn_ref[...] = diff / (std + 1e-5) - bpre_ref[...]

    tile = (
        jnp.dot(xn_ref[...], w_ref[...], preferred_element_type=jnp.float32)
        + benc_ref[...]
    )
    pre_ref[...] = tile
    pre_all[:, pl.ds(j * _TH, _TH)] = tile

    @pl.when(j == pl.num_programs(0) - 1)
    def _():
        v = pre_all[...]
        cpos = jnp.sum((v > 0.0).astype(jnp.int32), axis=1, keepdims=True)

        def body(_, carry):
            lo, hi = carry
            mid = lo + (hi - lo) // 2
            tm = lax.bitcast_convert_type(mid, jnp.float32)
            cnt = jnp.sum((v >= tm).astype(jnp.int32), axis=1, keepdims=True)
            ge = cnt >= _K
            return jnp.where(ge, mid, lo), jnp.where(ge, hi, mid)

        lo0 = jnp.zeros((_B, 1), jnp.int32)
        hi0 = jnp.full((_B, 1), 0x7F800001, jnp.int32)
        lo, _ = lax.fori_loop(0, 31, body, (lo0, hi0))
        t = lax.bitcast_convert_type(lo, jnp.float32)
        t_ref[...] = jnp.where(cpos > _K, t, 0.0)


def _popcount_scalar(m):
    pcv = plsc.all_reduce_population_count(m)
    return lax.squeeze(lax.slice(pcv, (0,), (1,)), (0,))


def _lane_splat(chunk, lane_idx_vec):
    dn = lax.GatherDimensionNumbers(
        offset_dims=(), collapsed_slice_dims=(0,), start_index_map=(0,))
    return lax.gather(chunk, lane_idx_vec[:, None], dn, (1,),
                      mode=lax.GatherScatterMode.PROMISE_IN_BOUNDS)


def _row_splat(ref_v, base, widv):
    lane = jnp.bitwise_and(widv, _L - 1)
    s0 = _lane_splat(ref_v[pl.ds(base, _L)], lane)
    s1 = _lane_splat(ref_v[pl.ds(base + _L, _L)], lane)
    return jnp.where(widv < _L, s0, s1)


def _sc_decode_body(pre_hbm, aux_hbm, bpre_hbm, wdec_hbm,
                    out_hbm,
                    row_v, aux_v, ei, si, acc,
                    rows0, rows1, sem0, sem1, sem2):
    wid = lax.axis_index("s") * _NC + lax.axis_index("c")
    d_row = pltpu.async_copy(pre_hbm.at[wid], row_v, sem0)
    d_aux = pltpu.async_copy(aux_hbm, aux_v, sem1)
    d_acc = pltpu.async_copy(bpre_hbm, acc, sem2)

    zi = jnp.zeros((_L,), jnp.int32)
    for c in range(_CAP // _L):
        si[pl.ds(c * _L, _L)] = zi
    laneiota = lax.iota(jnp.int32, _L)

    d_aux.wait()
    widv = jnp.zeros((_L,), jnp.int32) + wid
    ts = _row_splat(aux_v, 0, widv)
    pos_t = jnp.maximum(ts, 0.0)
    d_row.wait()

    def fast_body(i, nc):
        v = row_v[pl.ds(i * _L, _L)]
        m = jnp.logical_and(v >= pos_t, nc <= _K)
        plsc.store_compressed(si.at[pl.ds(jnp.minimum(nc, _K + _L), _L)],
                              laneiota + i * _L, mask=m)
        return nc + _popcount_scalar(v >= pos_t)

    nc = lax.fori_loop(0, _H // _L, fast_body, jnp.int32(0), unroll=8)

    def slow_path():
        def scan_body(i, carry):
            ns, ne = carry
            v = row_v[pl.ds(i * _L, _L)]
            idx = laneiota + i * _L
            m = v > pos_t
            plsc.store_compressed(si.at[pl.ds(ns, _L)], idx, mask=m)
            ns = ns + _popcount_scalar(m)
            meq = jnp.logical_and(v == pos_t, ne <= _K)
            plsc.store_compressed(ei.at[pl.ds(ne, _L)], idx, mask=meq)
            ne = ne + _popcount_scalar(meq)
            return ns, ne

        ns, ne = lax.fori_loop(0, _H // _L, scan_body,
                               (jnp.int32(0), jnp.int32(0)), unroll=4)
        need = jnp.minimum(_K - ns, ne)
        needv = jnp.zeros((_L,), jnp.int32) + need
        for c in range(_K // _L):
            keep = (laneiota + c * _L) < needv
            plsc.store_compressed(si.at[pl.ds(ns + c * _L, _L)],
                                  ei[pl.ds(c * _L, _L)], mask=keep)
        return ns + need

    total = lax.cond(nc == _K, lambda: jnp.int32(_K), slow_path)
    totalv = jnp.zeros((_L,), jnp.int32) + total

    bufs = [rows0, rows1]
    sems = [sem0, sem1]
    descs = [None] * _GRP
    descs[0] = pltpu.async_copy(
        wdec_hbm.at[si.at[pl.ds(0, _GS)]], bufs[0], sems[0])
    d_acc.wait()
    vchunk = None
    for g in range(_GRP):
        if g + 1 < _GRP:
            descs[g + 1] = pltpu.async_copy(
                wdec_hbm.at[si.at[pl.ds((g + 1) * _GS, _GS)]],
                bufs[(g + 1) % 2], sems[(g + 1) % 2])
        if g % 2 == 0:
            slot = laneiota + (g // 2) * _L
            ichunk = si[pl.ds((g // 2) * _L, _L)]
            vchunk = jnp.where(slot < totalv,
                               plsc.load_gather(row_v, [ichunk]), 0.0)
        descs[g].wait()
        cur = bufs[g % 2]
        vsp = [_lane_splat(vchunk, jnp.full((_L,), (g % 2) * _GS + r, jnp.int32))
               for r in range(_GS)]

        @plsc.parallel_loop(0, _D // _L, 1, unroll=8)
        def _mac(c):
            base = c * _L
            a = acc[pl.ds(base, _L)]
            for r in range(_GS):
                a = a + vsp[r] * cur[r, pl.ds(base, _L)]
            acc[pl.ds(base, _L)] = a

    mus = _row_splat(aux_v, _B, widv)
    sds = _row_splat(aux_v, 2 * _B, widv)

    @plsc.parallel_loop(0, _D // _L, 1, unroll=4)
    def _fin(c):
        base = c * _L
        acc[pl.ds(base, _L)] = acc[pl.ds(base, _L)] * sds + mus

    pltpu.sync_copy(acc, out_hbm.at[wid])


def kernel(x, w_enc, w_dec, b_enc, b_pre):
    pre, mu, std, t = pl.pallas_call(
        _encode_body,
        grid=(_H // _TH,),
        in_specs=[
            pl.BlockSpec((_B, _D), lambda j: (0, 0)),
            pl.BlockSpec((_D, _TH), lambda j: (0, j)),
            pl.BlockSpec((_TH,), lambda j: (j,)),
            pl.BlockSpec((_D,), lambda j: (0,)),
        ],
        out_specs=[
            pl.BlockSpec((_B, _TH), lambda j: (0, j)),
            pl.BlockSpec((_B, 1), lambda j: (0, 0)),
            pl.BlockSpec((_B, 1), lambda j: (0, 0)),
            pl.BlockSpec((_B, 1), lambda j: (0, 0)),
        ],
        out_shape=[
            jax.ShapeDtypeStruct((_B, _H), jnp.float32),
            jax.ShapeDtypeStruct((_B, 1), jnp.float32),
            jax.ShapeDtypeStruct((_B, 1), jnp.float32),
            jax.ShapeDtypeStruct((_B, 1), jnp.float32),
        ],
        scratch_shapes=[
            pltpu.VMEM((_B, _D), jnp.float32),
            pltpu.VMEM((_B, _H), jnp.float32),
        ],
    )(x, w_enc, b_enc, b_pre)

    sc_decode = pl.kernel(
        _sc_decode_body,
        out_type=jax.ShapeDtypeStruct((_B, _D), jnp.float32),
        mesh=plsc.VectorSubcoreMesh(core_axis_name="c", subcore_axis_name="s",
                                    num_cores=_NC, num_subcores=_NS),
        scratch_types=[
            pltpu.VMEM((_H,), jnp.float32),
            pltpu.VMEM((3 * _B,), jnp.float32),
            pltpu.VMEM((112,), jnp.int32),
            pltpu.VMEM((_CAP,), jnp.int32),
            pltpu.VMEM((_D,), jnp.float32),
            pltpu.VMEM((_GS, _D), jnp.float32),
            pltpu.VMEM((_GS, _D), jnp.float32),
            pltpu.SemaphoreType.DMA,
            pltpu.SemaphoreType.DMA,
            pltpu.SemaphoreType.DMA,
        ],
        compiler_params=pltpu.CompilerParams(needs_layout_passes=False),
    )
    aux = jnp.concatenate([t.reshape(_B), mu.reshape(_B), std.reshape(_B)])
    recons = sc_decode(pre, aux, b_pre, w_dec)

    num_dead = jnp.zeros((), jnp.int32)
    return (recons, num_dead)
